# baseline (speedup 1.0000x reference)
import functools

import jax
import jax.numpy as jnp
from jax import lax
from jax.experimental import pallas as pl
from jax.experimental.pallas import tpu as pltpu
from jax.experimental.pallas import tpu_sc as plsc

_NEG_TH = 0.3
_MSE_TH = 1.0
_K = 8
_L = 16
_NW = 16
_UNROLL = 8


def _merge_top16(top16, y):
    cdesc = lax.rev(jnp.sort(y), (0,))
    return jnp.sort(jnp.maximum(top16, cdesc))


def _merge_sorted(ta, tb):
    return jnp.sort(jnp.maximum(ta, lax.rev(tb, (0,))))


def _sc_body(rows_per_worker, num_chunks, wid_fn, pred_hbm, lab_hbm, out_hbm,
             pbufa, lbufa, pbufb, lbufb, obuf, sema, semb):
    wid = wid_fn()
    base_row = wid * rows_per_worker
    lanes = lax.iota(jnp.int32, _L)
    big_i32 = jnp.full((_L,), jnp.int32(2**31 - 1))

    def start_set(row0, pbuf, lbuf, sem):
        pltpu.make_async_copy(pred_hbm.at[pl.ds(row0, 2)], pbuf, sem).start()
        pltpu.make_async_copy(lab_hbm.at[pl.ds(row0, 2)], lbuf, sem).start()

    def wait_set(row0, pbuf, lbuf, sem):
        pltpu.make_async_copy(pred_hbm.at[pl.ds(row0, 2)], pbuf, sem).wait()
        pltpu.make_async_copy(lab_hbm.at[pl.ds(row0, 2)], lbuf, sem).wait()

    def process_row(pbuf, lbuf, rr, acc_top8):

        def chunk_body(j, tops):
            tops = list(tops)
            base = j * _UNROLL * _L
            ys = []
            for u in range(_UNROLL):
                sl = pl.ds(pl.multiple_of(base + u * _L, _L), _L)
                p = pbuf[rr, sl]
                l = lbuf[rr, sl]
                ys.append(jnp.where(l < _NEG_TH, jnp.maximum(p, 0.0), 0.0))
            for u in range(_UNROLL // 2):
                a = jnp.sort(ys[2 * u])
                b, _ = plsc.sort_key_val(ys[2 * u + 1], ys[2 * u + 1],
                                         descending=True)
                m = jnp.maximum(a, b)
                mdesc, _ = plsc.sort_key_val(m, m, descending=True)
                tops[u] = jnp.sort(jnp.maximum(tops[u], mdesc))
            return tuple(tops)

        tops0 = tuple(jnp.zeros((_L,), jnp.float32)
                      for _ in range(_UNROLL // 2))
        tops = lax.fori_loop(0, num_chunks // _UNROLL, chunk_body, tops0)

        tl = list(tops)
        while len(tl) > 1:
            tl = [_merge_sorted(tl[i], tl[i + 1]) for i in range(0, len(tl), 2)]
        return acc_top8 + jnp.where(lanes >= _L - _K, tl[0], 0.0)

    def process_pair(pbuf, lbuf, acc):
        acc = process_row(pbuf, lbuf, 0, acc)
        return process_row(pbuf, lbuf, 1, acc)

    start_set(base_row, pbufa, lbufa, sema)

    def quad_body(i, accs):
        r0 = base_row + i * 4
        start_set(r0 + 2, pbufb, lbufb, semb)
        wait_set(r0, pbufa, lbufa, sema)
        accs = process_pair(pbufa, lbufa, accs)

        @pl.when(i * 4 + 4 < rows_per_worker)
        def _():
            start_set(r0 + 4, pbufa, lbufa, sema)

        wait_set(r0 + 2, pbufb, lbufb, semb)
        accs = process_pair(pbufb, lbufb, accs)
        return accs

    z = jnp.zeros((_L,), jnp.float32)
    acc_top8 = lax.fori_loop(0, rows_per_worker // 4, quad_body, z)

    s_top8 = jnp.full((_L,), 0.0) + jnp.sum(acc_top8)
    obuf[...] = jnp.where(lanes == 0, s_top8, 0.0)
    pltpu.sync_copy(obuf, out_hbm.at[wid])


_TC_BLOCK = 128


def _tc_body(pred_ref, lab_ref, out_ref):
    p = pred_ref[...]
    l = lab_ref[...]
    r, hw = p.shape
    maxl = jnp.max(l, axis=1, keepdims=True)
    iota = lax.broadcasted_iota(jnp.int32, (r, hw), 1)
    big = jnp.int32(2**31 - 1)
    idx = jnp.min(jnp.where(l == maxl, iota, big), axis=1, keepdims=True)
    pa = jnp.sum(jnp.where(iota == idx, p, 0.0), axis=1, keepdims=True)
    validb = maxl > _NEG_TH
    pv_part = jnp.sum(jnp.where(validb, jnp.minimum(pa, 1.0), 0.0))
    valid_part = jnp.sum(jnp.where(validb, 1.0, 0.0))
    res = p - l
    sq = res * res
    sq_part = jnp.sum(jnp.where((sq > _MSE_TH) & (l < _NEG_TH), sq, 0.0))

    lanes2d = lax.broadcasted_iota(jnp.int32, (1, 128), 1)
    vec = jnp.where(lanes2d == 0, pv_part,
                    jnp.where(lanes2d == 1, valid_part,
                              jnp.where(lanes2d == 2, sq_part, 0.0)))

    @pl.when(pl.program_id(0) == 0)
    def _():
        out_ref[...] = jnp.zeros_like(out_ref)

    out_ref[...] += vec


@functools.partial(jax.jit, static_argnums=(2, 3))
def _run(pred, lab, rows, hw):
    rows_per_worker = rows // _NW
    num_chunks = hw // _L
    mesh = plsc.VectorSubcoreMesh(
        core_axis_name="c", subcore_axis_name="s", num_cores=1, num_subcores=16)
    wid_fn = lambda: lax.axis_index("s")
    body = functools.partial(_sc_body, rows_per_worker, num_chunks, wid_fn)
    parts = pl.kernel(
        body,
        out_type=jax.ShapeDtypeStruct((_NW, _L), jnp.float32),
        mesh=mesh,
        scratch_types=[
            pltpu.VMEM((2, hw), jnp.float32),
            pltpu.VMEM((2, hw), jnp.float32),
            pltpu.VMEM((2, hw), jnp.float32),
            pltpu.VMEM((2, hw), jnp.float32),
            pltpu.VMEM((_L,), jnp.float32),
            pltpu.SemaphoreType.DMA,
            pltpu.SemaphoreType.DMA,
        ],
        compiler_params=pltpu.CompilerParams(needs_layout_passes=False),
    )(pred, lab)

    tc_parts = pl.pallas_call(
        _tc_body,
        grid=(rows // _TC_BLOCK,),
        in_specs=[
            pl.BlockSpec((_TC_BLOCK, hw), lambda i: (i, 0)),
            pl.BlockSpec((_TC_BLOCK, hw), lambda i: (i, 0)),
        ],
        out_specs=pl.BlockSpec((1, 128), lambda i: (0, 0)),
        out_shape=jax.ShapeDtypeStruct((1, 128), jnp.float32),
    )(pred, lab)

    total_top8 = jnp.sum(parts[:, 0])
    total_pv = tc_parts[0, 0]
    total_valid = tc_parts[0, 1]
    total_sq = tc_parts[0, 2]
    n_valid = jnp.maximum(total_valid, 1.0)
    margin = 1.0 - total_pv / n_valid + total_top8 / (rows * _K)
    mse = total_sq / (rows * hw)
    return margin + mse


def kernel(prediction, label):
    rows = prediction.shape[0]
    hw = prediction.shape[-2] * prediction.shape[-1]
    pred = prediction.reshape(rows, hw)
    lab = label.reshape(rows, hw)
    assert rows % (_NW * 4) == 0 and rows % _TC_BLOCK == 0
    assert hw % (_L * _UNROLL) == 0 and hw % 128 == 0
    return _run(pred, lab, rows, hw)


# speedup vs baseline: 1.1515x; 1.1515x over previous
import functools

import jax
import jax.numpy as jnp
from jax import lax
from jax.experimental import pallas as pl
from jax.experimental.pallas import tpu as pltpu
from jax.experimental.pallas import tpu_sc as plsc

_NEG_TH = 0.3
_MSE_TH = 1.0
_K = 8
_L = 16
_NW = 32
_UNROLL = 8


def _merge_top16(top16, y):
    cdesc = lax.rev(jnp.sort(y), (0,))
    return jnp.sort(jnp.maximum(top16, cdesc))


def _merge_sorted(ta, tb):
    return jnp.sort(jnp.maximum(ta, lax.rev(tb, (0,))))


def _sc_body(rows_per_worker, num_chunks, wid_fn, pred_hbm, lab_hbm, out_hbm,
             pbufa, lbufa, pbufb, lbufb, obuf, sema, semb):
    wid = wid_fn()
    base_row = wid * rows_per_worker
    lanes = lax.iota(jnp.int32, _L)
    big_i32 = jnp.full((_L,), jnp.int32(2**31 - 1))

    def start_set(row0, pbuf, lbuf, sem):
        pltpu.make_async_copy(pred_hbm.at[pl.ds(row0, 2)], pbuf, sem).start()
        pltpu.make_async_copy(lab_hbm.at[pl.ds(row0, 2)], lbuf, sem).start()

    def wait_set(row0, pbuf, lbuf, sem):
        pltpu.make_async_copy(pred_hbm.at[pl.ds(row0, 2)], pbuf, sem).wait()
        pltpu.make_async_copy(lab_hbm.at[pl.ds(row0, 2)], lbuf, sem).wait()

    def process_row(pbuf, lbuf, rr, acc_top8):

        def chunk_body(j, tops):
            tops = list(tops)
            base = j * _UNROLL * _L
            ys = []
            for u in range(_UNROLL):
                sl = pl.ds(pl.multiple_of(base + u * _L, _L), _L)
                p = pbuf[rr, sl]
                l = lbuf[rr, sl]
                ys.append(jnp.where(l < _NEG_TH, jnp.maximum(p, 0.0), 0.0))
            for u in range(_UNROLL // 2):
                a = jnp.sort(ys[2 * u])
                b, _ = plsc.sort_key_val(ys[2 * u + 1], ys[2 * u + 1],
                                         descending=True)
                m = jnp.maximum(a, b)
                mdesc, _ = plsc.sort_key_val(m, m, descending=True)
                tops[u] = jnp.sort(jnp.maximum(tops[u], mdesc))
            return tuple(tops)

        tops0 = tuple(jnp.zeros((_L,), jnp.float32)
                      for _ in range(_UNROLL // 2))
        tops = lax.fori_loop(0, num_chunks // _UNROLL, chunk_body, tops0)

        tl = list(tops)
        while len(tl) > 1:
            tl = [_merge_sorted(tl[i], tl[i + 1]) for i in range(0, len(tl), 2)]
        return acc_top8 + jnp.where(lanes >= _L - _K, tl[0], 0.0)

    def process_pair(pbuf, lbuf, acc):
        acc = process_row(pbuf, lbuf, 0, acc)
        return process_row(pbuf, lbuf, 1, acc)

    start_set(base_row, pbufa, lbufa, sema)

    def quad_body(i, accs):
        r0 = base_row + i * 4
        start_set(r0 + 2, pbufb, lbufb, semb)
        wait_set(r0, pbufa, lbufa, sema)
        accs = process_pair(pbufa, lbufa, accs)

        @pl.when(i * 4 + 4 < rows_per_worker)
        def _():
            start_set(r0 + 4, pbufa, lbufa, sema)

        wait_set(r0 + 2, pbufb, lbufb, semb)
        accs = process_pair(pbufb, lbufb, accs)
        return accs

    z = jnp.zeros((_L,), jnp.float32)
    acc_top8 = lax.fori_loop(0, rows_per_worker // 4, quad_body, z)

    s_top8 = jnp.full((_L,), 0.0) + jnp.sum(acc_top8)
    obuf[...] = jnp.where(lanes == 0, s_top8, 0.0)
    pltpu.sync_copy(obuf, out_hbm.at[wid])


_TC_BLOCK = 128


def _tc_body(pred_ref, lab_ref, out_ref):
    p = pred_ref[...]
    l = lab_ref[...]
    r, hw = p.shape
    maxl = jnp.max(l, axis=1, keepdims=True)
    iota = lax.broadcasted_iota(jnp.int32, (r, hw), 1)
    big = jnp.int32(2**31 - 1)
    idx = jnp.min(jnp.where(l == maxl, iota, big), axis=1, keepdims=True)
    pa = jnp.sum(jnp.where(iota == idx, p, 0.0), axis=1, keepdims=True)
    validb = maxl > _NEG_TH
    pv_part = jnp.sum(jnp.where(validb, jnp.minimum(pa, 1.0), 0.0))
    valid_part = jnp.sum(jnp.where(validb, 1.0, 0.0))
    res = p - l
    sq = res * res
    sq_part = jnp.sum(jnp.where((sq > _MSE_TH) & (l < _NEG_TH), sq, 0.0))

    lanes2d = lax.broadcasted_iota(jnp.int32, (1, 128), 1)
    vec = jnp.where(lanes2d == 0, pv_part,
                    jnp.where(lanes2d == 1, valid_part,
                              jnp.where(lanes2d == 2, sq_part, 0.0)))

    @pl.when(pl.program_id(0) == 0)
    def _():
        out_ref[...] = jnp.zeros_like(out_ref)

    out_ref[...] += vec


@functools.partial(jax.jit, static_argnums=(2, 3))
def _run(pred, lab, rows, hw):
    rows_per_worker = rows // _NW
    num_chunks = hw // _L
    mesh = plsc.VectorSubcoreMesh(
        core_axis_name="c", subcore_axis_name="s", num_cores=2, num_subcores=16)
    wid_fn = lambda: lax.axis_index("s") * 2 + lax.axis_index("c")
    body = functools.partial(_sc_body, rows_per_worker, num_chunks, wid_fn)
    parts = pl.kernel(
        body,
        out_type=jax.ShapeDtypeStruct((_NW, _L), jnp.float32),
        mesh=mesh,
        scratch_types=[
            pltpu.VMEM((2, hw), jnp.float32),
            pltpu.VMEM((2, hw), jnp.float32),
            pltpu.VMEM((2, hw), jnp.float32),
            pltpu.VMEM((2, hw), jnp.float32),
            pltpu.VMEM((_L,), jnp.float32),
            pltpu.SemaphoreType.DMA,
            pltpu.SemaphoreType.DMA,
        ],
        compiler_params=pltpu.CompilerParams(needs_layout_passes=False),
        cost_estimate=pl.CostEstimate(
            flops=400_000_000, bytes_accessed=310_000_000, transcendentals=0),
    )(pred, lab)

    tc_parts = pl.pallas_call(
        _tc_body,
        grid=(rows // _TC_BLOCK,),
        in_specs=[
            pl.BlockSpec((_TC_BLOCK, hw), lambda i: (i, 0)),
            pl.BlockSpec((_TC_BLOCK, hw), lambda i: (i, 0)),
        ],
        out_specs=pl.BlockSpec((1, 128), lambda i: (0, 0)),
        out_shape=jax.ShapeDtypeStruct((1, 128), jnp.float32),
    )(pred, lab)

    total_top8 = jnp.sum(parts[:, 0])
    total_pv = tc_parts[0, 0]
    total_valid = tc_parts[0, 1]
    total_sq = tc_parts[0, 2]
    n_valid = jnp.maximum(total_valid, 1.0)
    margin = 1.0 - total_pv / n_valid + total_top8 / (rows * _K)
    mse = total_sq / (rows * hw)
    return margin + mse


def kernel(prediction, label):
    rows = prediction.shape[0]
    hw = prediction.shape[-2] * prediction.shape[-1]
    pred = prediction.reshape(rows, hw)
    lab = label.reshape(rows, hw)
    assert rows % (_NW * 4) == 0 and rows % _TC_BLOCK == 0
    assert hw % (_L * _UNROLL) == 0 and hw % 128 == 0
    return _run(pred, lab, rows, hw)


# speedup vs baseline: 1.1519x; 1.0003x over previous
import functools

import jax
import jax.numpy as jnp
from jax import lax
from jax.experimental import pallas as pl
from jax.experimental.pallas import tpu as pltpu
from jax.experimental.pallas import tpu_sc as plsc

_NEG_TH = 0.3
_MSE_TH = 1.0
_K = 8
_L = 16
_NW = 32
_UNROLL = 8


def _merge_sorted(ta, tb):
    return jnp.sort(jnp.maximum(ta, lax.rev(tb, (0,))))


def _sc_body(rows_per_worker, num_chunks, wid_fn, pred_hbm, lab_hbm, out_hbm,
             pbufa, lbufa, pbufb, lbufb, obuf, sema, semb):
    wid = wid_fn()
    base_row = wid * rows_per_worker
    lanes = lax.iota(jnp.int32, _L)

    def start_set(row0, pbuf, lbuf, sem):
        pltpu.make_async_copy(pred_hbm.at[pl.ds(row0, 2)], pbuf, sem).start()
        pltpu.make_async_copy(lab_hbm.at[pl.ds(row0, 2)], lbuf, sem).start()

    def wait_set(row0, pbuf, lbuf, sem):
        pltpu.make_async_copy(pred_hbm.at[pl.ds(row0, 2)], pbuf, sem).wait()
        pltpu.make_async_copy(lab_hbm.at[pl.ds(row0, 2)], lbuf, sem).wait()

    def process_row(pbuf, lbuf, rr, acc_top8):

        def chunk_body(j, tops):
            tops = list(tops)
            base = j * _UNROLL * _L
            ys = []
            for u in range(_UNROLL):
                sl = pl.ds(pl.multiple_of(base + u * _L, _L), _L)
                p = pbuf[rr, sl]
                l = lbuf[rr, sl]
                ys.append(jnp.where(l < _NEG_TH, jnp.maximum(p, 0.0), 0.0))
            for u in range(_UNROLL // 2):
                a = jnp.sort(ys[2 * u])
                b, _ = plsc.sort_key_val(ys[2 * u + 1], ys[2 * u + 1],
                                         descending=True)
                m = jnp.maximum(a, b)
                mdesc, _ = plsc.sort_key_val(m, m, descending=True)
                tops[u] = jnp.sort(jnp.maximum(tops[u], mdesc))
            return tuple(tops)

        tops0 = tuple(jnp.zeros((_L,), jnp.float32)
                      for _ in range(_UNROLL // 2))
        tops = lax.fori_loop(0, num_chunks // _UNROLL, chunk_body, tops0)

        tl = list(tops)
        while len(tl) > 1:
            tl = [_merge_sorted(tl[i], tl[i + 1]) for i in range(0, len(tl), 2)]
        return acc_top8 + jnp.where(lanes >= _L - _K, tl[0], 0.0)

    def process_pair(pbuf, lbuf, acc):
        acc = process_row(pbuf, lbuf, 0, acc)
        return process_row(pbuf, lbuf, 1, acc)

    start_set(base_row, pbufa, lbufa, sema)

    def quad_body(i, accs):
        r0 = base_row + i * 4
        start_set(r0 + 2, pbufb, lbufb, semb)
        wait_set(r0, pbufa, lbufa, sema)
        accs = process_pair(pbufa, lbufa, accs)

        @pl.when(i * 4 + 4 < rows_per_worker)
        def _():
            start_set(r0 + 4, pbufa, lbufa, sema)

        wait_set(r0 + 2, pbufb, lbufb, semb)
        accs = process_pair(pbufb, lbufb, accs)
        return accs

    z = jnp.zeros((_L,), jnp.float32)
    acc_top8 = lax.fori_loop(0, rows_per_worker // 4, quad_body, z)

    s_top8 = jnp.full((_L,), 0.0) + jnp.sum(acc_top8)
    obuf[...] = jnp.where(lanes == 0, s_top8, 0.0)
    pltpu.sync_copy(obuf, out_hbm.at[wid])


_TC_BLOCK = 128


def _tc_body(pred_ref, lab_ref, out_ref):
    p = pred_ref[...]
    l = lab_ref[...]
    r, hw = p.shape
    maxl = jnp.max(l, axis=1, keepdims=True)
    iota = lax.broadcasted_iota(jnp.int32, (r, hw), 1)
    big = jnp.int32(2**31 - 1)
    idx = jnp.min(jnp.where(l == maxl, iota, big), axis=1, keepdims=True)
    pa = jnp.sum(jnp.where(iota == idx, p, 0.0), axis=1, keepdims=True)
    validb = maxl > _NEG_TH
    pv_part = jnp.sum(jnp.where(validb, jnp.minimum(pa, 1.0), 0.0))
    valid_part = jnp.sum(jnp.where(validb, 1.0, 0.0))
    res = p - l
    sq = res * res
    sq_part = jnp.sum(jnp.where((sq > _MSE_TH) & (l < _NEG_TH), sq, 0.0))

    lanes2d = lax.broadcasted_iota(jnp.int32, (1, 128), 1)
    vec = jnp.where(lanes2d == 0, pv_part,
                    jnp.where(lanes2d == 1, valid_part,
                              jnp.where(lanes2d == 2, sq_part, 0.0)))

    @pl.when(pl.program_id(0) == 0)
    def _():
        out_ref[...] = jnp.zeros_like(out_ref)

    out_ref[...] += vec


@functools.partial(jax.jit, static_argnums=(2, 3))
def _run(pred, lab, rows, hw):
    rows_per_worker = rows // _NW
    num_chunks = hw // _L
    mesh = plsc.VectorSubcoreMesh(
        core_axis_name="c", subcore_axis_name="s", num_cores=2, num_subcores=16)
    wid_fn = lambda: lax.axis_index("s") * 2 + lax.axis_index("c")
    body = functools.partial(_sc_body, rows_per_worker, num_chunks, wid_fn)
    parts = pl.kernel(
        body,
        out_type=jax.ShapeDtypeStruct((_NW, _L), jnp.float32),
        mesh=mesh,
        scratch_types=[
            pltpu.VMEM((2, hw), jnp.float32),
            pltpu.VMEM((2, hw), jnp.float32),
            pltpu.VMEM((2, hw), jnp.float32),
            pltpu.VMEM((2, hw), jnp.float32),
            pltpu.VMEM((_L,), jnp.float32),
            pltpu.SemaphoreType.DMA,
            pltpu.SemaphoreType.DMA,
        ],
        compiler_params=pltpu.CompilerParams(needs_layout_passes=False),
        cost_estimate=pl.CostEstimate(
            flops=400_000_000, bytes_accessed=310_000_000, transcendentals=0),
    )(pred, lab)

    tc_parts = pl.pallas_call(
        _tc_body,
        grid=(rows // _TC_BLOCK,),
        in_specs=[
            pl.BlockSpec((_TC_BLOCK, hw), lambda i: (i, 0)),
            pl.BlockSpec((_TC_BLOCK, hw), lambda i: (i, 0)),
        ],
        out_specs=pl.BlockSpec((1, 128), lambda i: (0, 0)),
        out_shape=jax.ShapeDtypeStruct((1, 128), jnp.float32),
    )(pred, lab)

    total_top8 = jnp.sum(parts[:, 0])
    total_pv = tc_parts[0, 0]
    total_valid = tc_parts[0, 1]
    total_sq = tc_parts[0, 2]
    n_valid = jnp.maximum(total_valid, 1.0)
    margin = 1.0 - total_pv / n_valid + total_top8 / (rows * _K)
    mse = total_sq / (rows * hw)
    return margin + mse


def kernel(prediction, label):
    rows = prediction.shape[0]
    hw = prediction.shape[-2] * prediction.shape[-1]
    pred = prediction.reshape(rows, hw)
    lab = label.reshape(rows, hw)
    assert rows % (_NW * 4) == 0 and rows % _TC_BLOCK == 0
    assert hw % (_L * _UNROLL) == 0 and hw % 128 == 0
    return _run(pred, lab, rows, hw)
